# baseline (speedup 1.0000x reference)
import jax
import jax.numpy as jnp
from jax.experimental import pallas as pl
from jax.experimental.pallas import tpu as pltpu

SEQ_LEN = 4096
EMB = 128
_NCHUNK = 2
_CHUNK_ROWS = SEQ_LEN // _NCHUNK


def _copy_body(emb_hbm, out_hbm, bufs, sem_in, sem_out):
    ins = [
        pltpu.make_async_copy(
            emb_hbm.at[pl.ds(i * _CHUNK_ROWS, _CHUNK_ROWS)],
            bufs.at[i],
            sem_in.at[i],
        )
        for i in range(_NCHUNK)
    ]
    outs = [
        pltpu.make_async_copy(
            bufs.at[i],
            out_hbm.at[pl.ds(i * _CHUNK_ROWS, _CHUNK_ROWS)],
            sem_out.at[i],
        )
        for i in range(_NCHUNK)
    ]
    for c in ins:
        c.start()
    for i in range(_NCHUNK):
        ins[i].wait()
        outs[i].start()
    for c in outs:
        c.wait()


def kernel(inputs, embedding_matrix):
    del inputs
    return pl.pallas_call(
        _copy_body,
        in_specs=[pl.BlockSpec(memory_space=pl.ANY)],
        out_specs=pl.BlockSpec(memory_space=pl.ANY),
        scratch_shapes=[
            pltpu.VMEM((_NCHUNK, _CHUNK_ROWS, EMB), jnp.float32),
            pltpu.SemaphoreType.DMA((_NCHUNK,)),
            pltpu.SemaphoreType.DMA((_NCHUNK,)),
        ],
        out_shape=jax.ShapeDtypeStruct((SEQ_LEN, EMB), jnp.float32),
    )(embedding_matrix)
